# Initial kernel scaffold; baseline (speedup 1.0000x reference)
import jax
import jax.numpy as jnp
from jax.experimental import pallas as pl


def kernel(x, edge_index, W1, b1, W2, b2):
    raise NotImplementedError("write your pallas kernel here")


# speedup vs baseline: 5.2973x; 5.2973x over previous
import functools

import jax
import jax.numpy as jnp
from jax import lax
from jax.experimental import pallas as pl
from jax.experimental.pallas import tpu as pltpu
from jax.experimental.pallas import tpu_sc as plsc

NC, NS = 2, 16
NW = NC * NS
CHUNK = 128
LANES = 16


def _mesh():
    return plsc.VectorSubcoreMesh(
        core_axis_name="c", subcore_axis_name="s", num_cores=NC, num_subcores=NS
    )


def _make_prep(e_pad, deg_r, n_dummy):
    epw = e_pad // NW
    nch = epw // CHUNK
    rpt = deg_r // NS

    @functools.partial(
        pl.kernel,
        out_type=(
            jax.ShapeDtypeStruct((e_pad,), jnp.int32),
            jax.ShapeDtypeStruct((NC * deg_r,), jnp.float32),
        ),
        mesh=_mesh(),
        scratch_types=[
            pltpu.VMEM((CHUNK,), jnp.int32),
            pltpu.VMEM((CHUNK,), jnp.int32),
            pltpu.VMEM((CHUNK,), jnp.int32),
            pltpu.VMEM((CHUNK,), jnp.float32),
            pltpu.VMEM((rpt,), jnp.float32),
            pltpu.VMEM_SHARED((deg_r,), jnp.float32),
        ],
    )
    def prep(src_hbm, dst_hbm, dstadj_hbm, degp_hbm, src_v, dst_v, adj_v,
             val_v, zb_v, deg_sh):
        cid = lax.axis_index("c")
        sid = lax.axis_index("s")
        wid = cid * NS + sid

        for i in range(rpt // LANES):
            zb_v[pl.ds(i * LANES, LANES)] = jnp.zeros((LANES,), jnp.float32)
        pltpu.sync_copy(zb_v, deg_sh.at[pl.ds(pl.multiple_of(sid * rpt, 8), rpt)])
        plsc.subcore_barrier()

        base = wid * epw

        def body(c, carry):
            off = pl.multiple_of(base + c * CHUNK, 8)
            pltpu.sync_copy(src_hbm.at[pl.ds(off, CHUNK)], src_v)
            pltpu.sync_copy(dst_hbm.at[pl.ds(off, CHUNK)], dst_v)
            for j in range(CHUNK // LANES):
                s = src_v[pl.ds(j * LANES, LANES)]
                d = dst_v[pl.ds(j * LANES, LANES)]
                m = s != d
                adj_v[pl.ds(j * LANES, LANES)] = jnp.where(
                    m, d, jnp.full((LANES,), n_dummy, jnp.int32))
                val_v[pl.ds(j * LANES, LANES)] = jnp.where(
                    m, jnp.full((LANES,), 1.0, jnp.float32),
                    jnp.zeros((LANES,), jnp.float32))
            pltpu.sync_copy(adj_v, dstadj_hbm.at[pl.ds(off, CHUNK)])
            pltpu.sync_copy(val_v, deg_sh.at[src_v], add=True)
            return carry

        lax.fori_loop(0, nch, body, 0)
        plsc.subcore_barrier()
        row = pl.multiple_of(cid * deg_r + sid * rpt, 8)
        pltpu.sync_copy(deg_sh.at[pl.ds(pl.multiple_of(sid * rpt, 8), rpt)],
                        degp_hbm.at[pl.ds(row, rpt)])

    return prep


def _make_prop(n, f, e_pad, acc_r):
    epw = e_pad // NW
    nch = epw // CHUNK
    rpt = acc_r // NS
    zrows = 16

    @functools.partial(
        pl.kernel,
        out_type=jax.ShapeDtypeStruct((NC * acc_r, f), jnp.float32),
        mesh=_mesh(),
        scratch_types=[
            pltpu.VMEM((CHUNK,), jnp.int32),
            pltpu.VMEM((CHUNK,), jnp.int32),
            pltpu.VMEM((CHUNK, f), jnp.float32),
            pltpu.VMEM((zrows, f), jnp.float32),
            pltpu.VMEM_SHARED((acc_r, f), jnp.float32),
            pltpu.SemaphoreType.DMA,
        ],
    )
    def prop(g_hbm, src_hbm, dst_hbm, out_hbm, src_v, dst_v, rows_v, zb_v,
             acc_sh, sem):
        cid = lax.axis_index("c")
        sid = lax.axis_index("s")
        wid = cid * NS + sid

        for r in range(zrows):
            for j in range(f // LANES):
                zb_v[r, pl.ds(j * LANES, LANES)] = jnp.zeros((LANES,),
                                                             jnp.float32)
        for k in range(rpt // zrows):
            pltpu.sync_copy(
                zb_v,
                acc_sh.at[pl.ds(pl.multiple_of(sid * rpt + k * zrows, 8),
                                zrows)])
        plsc.subcore_barrier()

        base = wid * epw

        def body(c, carry):
            off = pl.multiple_of(base + c * CHUNK, 8)
            pltpu.sync_copy(src_hbm.at[pl.ds(off, CHUNK)], src_v)
            pltpu.sync_copy(dst_hbm.at[pl.ds(off, CHUNK)], dst_v)
            pltpu.async_copy(g_hbm.at[src_v], rows_v, sem).wait()
            pltpu.sync_copy(rows_v, acc_sh.at[dst_v], add=True)
            return carry

        lax.fori_loop(0, nch, body, 0)
        plsc.subcore_barrier()
        row = pl.multiple_of(cid * acc_r + sid * rpt, 8)
        pltpu.sync_copy(acc_sh.at[pl.ds(pl.multiple_of(sid * rpt, 8), rpt)],
                        out_hbm.at[pl.ds(row, rpt)])

    return prop


def _tc_prep_call(degp, x, br):
    n, f = x.shape
    deg_r = degp.shape[1]

    def body(degp_ref, x_ref, dinv_ref, g0_ref):
        deg = degp_ref[0] + degp_ref[1]
        dinv = jnp.where(deg > 0.0, lax.rsqrt(jnp.maximum(deg, 1.0)), 0.0)
        dinv_ref[...] = dinv
        g0_ref[...] = x_ref[...] * dinv

    return pl.pallas_call(
        body,
        grid=(n // br,),
        in_specs=[
            pl.BlockSpec((2, br, 1), lambda i: (0, i, 0)),
            pl.BlockSpec((br, f), lambda i: (i, 0)),
        ],
        out_specs=[
            pl.BlockSpec((br, 1), lambda i: (i, 0)),
            pl.BlockSpec((br, f), lambda i: (i, 0)),
        ],
        out_shape=[
            jax.ShapeDtypeStruct((n, 1), jnp.float32),
            jax.ShapeDtypeStruct((n, f), jnp.float32),
        ],
    )(degp, x)


def _tc_combine_call(p, dinv, br):
    _, acc_r, f = p.shape
    n = dinv.shape[0]

    def body(p_ref, dinv_ref, t_ref, g_ref):
        s = p_ref[0] + p_ref[1]
        dinv = dinv_ref[...]
        t = -(dinv * s)
        t_ref[...] = t
        g_ref[...] = dinv * t

    return pl.pallas_call(
        body,
        grid=(n // br,),
        in_specs=[
            pl.BlockSpec((2, br, f), lambda i: (0, i, 0)),
            pl.BlockSpec((br, 1), lambda i: (i, 0)),
        ],
        out_specs=[
            pl.BlockSpec((br, f), lambda i: (i, 0)),
            pl.BlockSpec((br, f), lambda i: (i, 0)),
        ],
        out_shape=[
            jax.ShapeDtypeStruct((n, f), jnp.float32),
            jax.ShapeDtypeStruct((n, f), jnp.float32),
        ],
    )(p, dinv)


def _tc_layer_call(t0, t1, q, dinv, W, b, br, relu_and_scale):
    n, f_in = t0.shape
    f_out = W.shape[2]

    def body(t0_ref, t1_ref, q_ref, dinv_ref, w_ref, b_ref, *out_refs):
        t0 = t0_ref[...]
        dinv = dinv_ref[...]
        t2 = -2.0 * (dinv * (q_ref[0] + q_ref[1])) - t0
        acc = jnp.dot(t0, w_ref[0], preferred_element_type=jnp.float32)
        acc += jnp.dot(t1_ref[...], w_ref[1], preferred_element_type=jnp.float32)
        acc += jnp.dot(t2, w_ref[2], preferred_element_type=jnp.float32)
        acc += b_ref[...]
        if relu_and_scale:
            h = jnp.maximum(acc, 0.0)
            out_refs[0][...] = h
            out_refs[1][...] = dinv * h
        else:
            out_refs[0][...] = acc

    out_shape = [jax.ShapeDtypeStruct((n, f_out), jnp.float32)]
    out_specs = [pl.BlockSpec((br, f_out), lambda i: (i, 0))]
    if relu_and_scale:
        out_shape.append(jax.ShapeDtypeStruct((n, f_out), jnp.float32))
        out_specs.append(pl.BlockSpec((br, f_out), lambda i: (i, 0)))

    return pl.pallas_call(
        body,
        grid=(n // br,),
        in_specs=[
            pl.BlockSpec((br, f_in), lambda i: (i, 0)),
            pl.BlockSpec((br, f_in), lambda i: (i, 0)),
            pl.BlockSpec((2, br, f_in), lambda i: (0, i, 0)),
            pl.BlockSpec((br, 1), lambda i: (i, 0)),
            pl.BlockSpec(W.shape, lambda i: (0, 0, 0)),
            pl.BlockSpec((1, f_out), lambda i: (0, 0)),
        ],
        out_specs=out_specs,
        out_shape=out_shape,
    )(t0, t1, q, dinv, W, b)


def kernel(x, edge_index, W1, b1, W2, b2):
    n, f = x.shape
    e = edge_index.shape[1]

    e_pad = ((e + NW * CHUNK - 1) // (NW * CHUNK)) * (NW * CHUNK)
    pad = e_pad - e
    src = jnp.concatenate([edge_index[0], jnp.zeros((pad,), jnp.int32)])
    dst = jnp.concatenate([edge_index[1], jnp.zeros((pad,), jnp.int32)])

    acc_r = ((n + 1 + NS * 16 - 1) // (NS * 16)) * (NS * 16)
    deg_r = acc_r
    br = 2000 if n % 2000 == 0 else 8

    prep = _make_prep(e_pad, deg_r, n)
    prop = _make_prop(n, f, e_pad, acc_r)

    dst_adj, degp = prep(src, dst)
    degp = degp.reshape(NC, deg_r, 1)

    dinv, g0 = _tc_prep_call(degp, x, br)

    p = prop(g0, src, dst_adj).reshape(NC, acc_r, f)
    t1, g1 = _tc_combine_call(p, dinv, br)
    q = prop(g1, src, dst_adj).reshape(NC, acc_r, f)
    h, gh = _tc_layer_call(x, t1, q, dinv, W1, b1.reshape(1, -1), br,
                           relu_and_scale=True)

    p2 = prop(gh, src, dst_adj).reshape(NC, acc_r, f)
    u1, g3 = _tc_combine_call(p2, dinv, br)
    q2 = prop(g3, src, dst_adj).reshape(NC, acc_r, f)
    out = _tc_layer_call(h, u1, q2, dinv, W2, b2.reshape(1, -1), br,
                         relu_and_scale=False)[0]

    return (out, edge_index)
